# baseline (speedup 1.0000x reference)
import jax
import jax.numpy as jnp
from jax.experimental import pallas as pl


def _copy_body(t_ref, o_ref):
    o_ref[...] = t_ref[...]


def kernel(input_ids, table):
    del input_ids
    S, H = table.shape
    blocks = 16
    out = pl.pallas_call(
        _copy_body,
        grid=(blocks,),
        in_specs=[pl.BlockSpec((S // blocks, H), lambda i: (i, 0))],
        out_specs=pl.BlockSpec((S // blocks, H), lambda i: (i, 0)),
        out_shape=jax.ShapeDtypeStruct((S, H), table.dtype),
    )(table)
    return out[None]


# speedup vs baseline: 1.7434x; 1.7434x over previous
import jax
import jax.numpy as jnp
from jax.experimental import pallas as pl


def _copy_body(t_ref, o_ref):
    o_ref[...] = t_ref[...]


def kernel(input_ids, table):
    del input_ids
    S, H = table.shape
    blocks = 4
    out = pl.pallas_call(
        _copy_body,
        grid=(blocks,),
        in_specs=[pl.BlockSpec((S // blocks, H), lambda i: (i, 0))],
        out_specs=pl.BlockSpec((S // blocks, H), lambda i: (i, 0)),
        out_shape=jax.ShapeDtypeStruct((S, H), table.dtype),
    )(table)
    return out[None]


# speedup vs baseline: 2.1426x; 1.2289x over previous
import jax
import jax.numpy as jnp
from jax.experimental import pallas as pl


def _copy_body(t_ref, o_ref):
    o_ref[...] = t_ref[...]


def kernel(input_ids, table):
    del input_ids
    S, H = table.shape
    blocks = 2
    out = pl.pallas_call(
        _copy_body,
        grid=(blocks,),
        in_specs=[pl.BlockSpec((S // blocks, H), lambda i: (i, 0))],
        out_specs=pl.BlockSpec((S // blocks, H), lambda i: (i, 0)),
        out_shape=jax.ShapeDtypeStruct((S, H), table.dtype),
    )(table)
    return out[None]
